# baseline (speedup 1.0000x reference)
import functools

import jax
import jax.numpy as jnp
from jax import lax
from jax.experimental import pallas as pl
from jax.experimental.pallas import tpu as pltpu

E_BLK = 128


def _kernel_body(n, n_pad, n_blk,
                 x_ref, mu_ref, var_ref, wsub_ref, ssub_ref,
                 wl_ref, bl_ref, wr_ref, br_ref, att_ref,
                 src_ref, dst_ref,
                 ll_ref, ce_ref, p_ref,
                 ex_ref, den_ref):
    f32 = jnp.float32
    x = x_ref[...]
    mu = mu_ref[...]
    var = var_ref[...]
    W = wsub_ref[...]
    S = ssub_ref[...]

    wmax = jnp.max(W, axis=1, keepdims=True)
    we = jnp.exp(W - wmax)
    P = we / jnp.sum(we, axis=1, keepdims=True)
    p_ref[...] = P
    logP = jnp.log(P + 1e-8)

    inv_var = 1.0 / var
    A = lax.dot_general(x * x, inv_var, (((1,), (1,)), ((), ())),
                        preferred_element_type=f32)
    B = lax.dot_general(x, mu * inv_var, (((1,), (1,)), ((), ())),
                        preferred_element_type=f32)
    Cc = jnp.sum(mu * mu * inv_var, axis=1)[None, :]
    F = -0.5 * (A - 2.0 * S * B + (S * S) * Cc)
    rmask = (lax.broadcasted_iota(jnp.int32, (n_pad, 1), 0) < n).astype(f32)
    ll_ref[...] = jnp.reshape(jnp.sum(P * F * rmask) / n, (1, 1))

    xl = lax.dot_general(x, wl_ref[...], (((1,), (0,)), ((), ())),
                         preferred_element_type=f32) + bl_ref[...]
    xr = lax.dot_general(x, wr_ref[...], (((1,), (0,)), ((), ())),
                         preferred_element_type=f32) + br_ref[...]
    att = att_ref[...]

    den_ref[...] = jnp.zeros_like(den_ref)

    def pass1(i, carry):
        s = jnp.reshape(src_ref[pl.ds(i, 1), :], (E_BLK, 1))
        d = jnp.reshape(dst_ref[pl.ds(i, 1), :], (E_BLK, 1))
        cols = lax.broadcasted_iota(jnp.int32, (E_BLK, n_pad), 1)
        Os = (cols == s).astype(f32)
        Od = (cols == d).astype(f32)
        g = (lax.dot_general(Os, xl, (((1,), (0,)), ((), ())),
                             preferred_element_type=f32)
             + lax.dot_general(Od, xr, (((1,), (0,)), ((), ())),
                               preferred_element_type=f32))
        v = jnp.where(g > 0, g, 0.2 * g)
        e = lax.dot_general(v, att, (((1,), (0,)), ((), ())),
                            preferred_element_type=f32)
        ex = jnp.exp(e)
        ex_ref[pl.ds(i, 1), :] = jnp.reshape(ex, (1, E_BLK))
        den_ref[...] += lax.dot_general(Od, ex, (((0,), (0,)), ((), ())),
                                        preferred_element_type=f32)
        return carry

    lax.fori_loop(0, n_blk, pass1, 0, unroll=False)
    den = den_ref[...]

    def pass2(i, acc):
        s = jnp.reshape(src_ref[pl.ds(i, 1), :], (E_BLK, 1))
        d = jnp.reshape(dst_ref[pl.ds(i, 1), :], (E_BLK, 1))
        cols = lax.broadcasted_iota(jnp.int32, (E_BLK, n_pad), 1)
        Os = (cols == s).astype(f32)
        Od = (cols == d).astype(f32)
        Ps = lax.dot_general(Os, P, (((1,), (0,)), ((), ())),
                             preferred_element_type=f32)
        lPd = lax.dot_general(Od, logP, (((1,), (0,)), ((), ())),
                              preferred_element_type=f32)
        q = jnp.sum(Ps * lPd, axis=1, keepdims=True)
        dd = lax.dot_general(Od, den, (((1,), (0,)), ((), ())),
                             preferred_element_type=f32)
        ex = jnp.reshape(ex_ref[pl.ds(i, 1), :], (E_BLK, 1))
        return acc + jnp.sum(ex * q / dd)

    tot = lax.fori_loop(0, n_blk, pass2, f32(0.0), unroll=False)
    ce_ref[...] = jnp.reshape(-tot / n, (1, 1))


def kernel(x_sub, Mu, Var, edge_index_sub, subset_idx, W, S, Wl, bl, Wr, br, att):
    n, g = x_sub.shape
    c = Mu.shape[0]
    h = Wl.shape[1]
    e = edge_index_sub.shape[1]
    n_pad = ((n + 127) // 128) * 128
    n_blk = e // E_BLK

    W_sub = jnp.take(W, subset_idx, axis=0)
    S_sub = jnp.take(S, subset_idx, axis=0)
    pad = n_pad - n
    x_p = jnp.pad(x_sub, ((0, pad), (0, 0)))
    W_p = jnp.pad(W_sub, ((0, pad), (0, 0)))
    S_p = jnp.pad(S_sub, ((0, pad), (0, 0)), constant_values=1.0)
    src = edge_index_sub[0].reshape(n_blk, E_BLK)
    dst = edge_index_sub[1].reshape(n_blk, E_BLK)

    body = functools.partial(_kernel_body, n, n_pad, n_blk)
    ll, ce, P_pad = pl.pallas_call(
        body,
        out_shape=[
            jax.ShapeDtypeStruct((1, 1), jnp.float32),
            jax.ShapeDtypeStruct((1, 1), jnp.float32),
            jax.ShapeDtypeStruct((n_pad, c), jnp.float32),
        ],
        scratch_shapes=[
            pltpu.VMEM((n_blk, E_BLK), jnp.float32),
            pltpu.VMEM((n_pad, 1), jnp.float32),
        ],
    )(x_p, Mu, Var, W_p, S_p,
      Wl, bl.reshape(1, h), Wr, br.reshape(1, h), att.reshape(h, 1),
      src, dst)

    return ll[0, 0], ce[0, 0], P_pad[:n]


# speedup vs baseline: 1.5068x; 1.5068x over previous
import functools

import jax
import jax.numpy as jnp
from jax import lax
from jax.experimental import pallas as pl
from jax.experimental.pallas import tpu as pltpu

E_BLK = 128


def _kernel_body(n, n_pad, n_blk,
                 x_ref, mu_ref, var_ref, wsub_ref, ssub_ref,
                 wl_ref, bl_ref, wr_ref, br_ref, att_ref,
                 src_ref, dst_ref,
                 ll_ref, ce_ref, p_ref,
                 acc_ref):
    f32 = jnp.float32
    x = x_ref[...]
    mu = mu_ref[...]
    var = var_ref[...]
    W = wsub_ref[...]
    S = ssub_ref[...]

    wmax = jnp.max(W, axis=1, keepdims=True)
    we = jnp.exp(W - wmax)
    P = we / jnp.sum(we, axis=1, keepdims=True)
    p_ref[...] = P
    logP = jnp.log(P + 1e-8)

    inv_var = 1.0 / var
    A = lax.dot_general(x * x, inv_var, (((1,), (1,)), ((), ())),
                        preferred_element_type=f32)
    B = lax.dot_general(x, mu * inv_var, (((1,), (1,)), ((), ())),
                        preferred_element_type=f32)
    Cc = jnp.sum(mu * mu * inv_var, axis=1)[None, :]
    F = -0.5 * (A - 2.0 * S * B + (S * S) * Cc)
    rmask = (lax.broadcasted_iota(jnp.int32, (n_pad, 1), 0) < n).astype(f32)
    ll_ref[...] = jnp.reshape(jnp.sum(P * F * rmask) / n, (1, 1))

    xl = lax.dot_general(x, wl_ref[...], (((1,), (0,)), ((), ())),
                         preferred_element_type=f32) + bl_ref[...]
    xr = lax.dot_general(x, wr_ref[...], (((1,), (0,)), ((), ())),
                         preferred_element_type=f32) + br_ref[...]
    att = att_ref[...]

    acc_ref[...] = jnp.zeros_like(acc_ref)

    def pass1(i, carry):
        s = jnp.reshape(src_ref[pl.ds(i, 1), :], (E_BLK, 1))
        d = jnp.reshape(dst_ref[pl.ds(i, 1), :], (E_BLK, 1))
        cols = lax.broadcasted_iota(jnp.int32, (E_BLK, n_pad), 1)
        Os = (cols == s).astype(f32)
        Od = (cols == d).astype(f32)
        g = (lax.dot_general(Os, xl, (((1,), (0,)), ((), ())),
                             preferred_element_type=f32)
             + lax.dot_general(Od, xr, (((1,), (0,)), ((), ())),
                               preferred_element_type=f32))
        v = jnp.where(g > 0, g, 0.2 * g)
        e = lax.dot_general(v, att, (((1,), (0,)), ((), ())),
                            preferred_element_type=f32)
        ex = jnp.exp(e)
        Ps = lax.dot_general(Os, P, (((1,), (0,)), ((), ())),
                             preferred_element_type=f32)
        upd = jnp.concatenate([ex, ex * Ps], axis=1)
        acc_ref[...] += lax.dot_general(Od, upd, (((0,), (0,)), ((), ())),
                                        preferred_element_type=f32)
        return carry

    lax.fori_loop(0, n_blk, pass1, 0, unroll=False)
    acc = acc_ref[...]
    den = acc[:, 0:1]
    M = acc[:, 1:]
    num = jnp.sum(logP * M, axis=1, keepdims=True)
    ce_contrib = jnp.where(den > 0, num / den, 0.0)
    ce_ref[...] = jnp.reshape(-jnp.sum(ce_contrib) / n, (1, 1))


def kernel(x_sub, Mu, Var, edge_index_sub, subset_idx, W, S, Wl, bl, Wr, br, att):
    n, g = x_sub.shape
    c = Mu.shape[0]
    h = Wl.shape[1]
    e = edge_index_sub.shape[1]
    n_pad = ((n + 127) // 128) * 128
    n_blk = e // E_BLK

    W_sub = jnp.take(W, subset_idx, axis=0)
    S_sub = jnp.take(S, subset_idx, axis=0)
    pad = n_pad - n
    x_p = jnp.pad(x_sub, ((0, pad), (0, 0)))
    W_p = jnp.pad(W_sub, ((0, pad), (0, 0)))
    S_p = jnp.pad(S_sub, ((0, pad), (0, 0)), constant_values=1.0)
    src = edge_index_sub[0].reshape(n_blk, E_BLK)
    dst = edge_index_sub[1].reshape(n_blk, E_BLK)

    body = functools.partial(_kernel_body, n, n_pad, n_blk)
    ll, ce, P_pad = pl.pallas_call(
        body,
        out_shape=[
            jax.ShapeDtypeStruct((1, 1), jnp.float32),
            jax.ShapeDtypeStruct((1, 1), jnp.float32),
            jax.ShapeDtypeStruct((n_pad, c), jnp.float32),
        ],
        scratch_shapes=[
            pltpu.VMEM((n_pad, 1 + c), jnp.float32),
        ],
    )(x_p, Mu, Var, W_p, S_p,
      Wl, bl.reshape(1, h), Wr, br.reshape(1, h), att.reshape(h, 1),
      src, dst)

    return ll[0, 0], ce[0, 0], P_pad[:n]
